# baseline (speedup 1.0000x reference)
import functools

import jax
import jax.numpy as jnp
from jax import lax
from jax.experimental import pallas as pl
from jax.experimental.pallas import tpu as pltpu
from jax.experimental.pallas import tpu_sc as plsc

N_NODES = 10000
N_EDGES = 320000
D = 128

NC = 2
NS = 16
NW = NC * NS
EPW = N_EDGES // NW
CH = 80
NCHUNK = EPW // CH
ROWS_PER_TILE = N_NODES // NS
ZROWS = 25
DEN_STRIDE = 10240


def _prep_body(feat_ref, mask_ref, wt_ref, attn_ref, edge_ref,
               g_ref, en_ref, edge_out_ref):
    hm = feat_ref[...] * mask_ref[...]
    h = jnp.dot(hm, wt_ref[...], preferred_element_type=jnp.float32)
    s = jnp.sum(h * attn_ref[...], axis=1, keepdims=True)
    s = jnp.where(s > 0, s, 0.01 * s)
    en = jnp.exp(s)
    g_ref[...] = h * en
    en_ref[...] = en[:, 0]
    edge_out_ref[pl.ds(0, N_EDGES)] = edge_ref[0]
    edge_out_ref[pl.ds(N_EDGES, N_EDGES)] = edge_ref[1]


def _prep(feat, mask_row, wt, attn_row, edge_index):
    return pl.pallas_call(
        _prep_body,
        out_shape=[
            jax.ShapeDtypeStruct((N_NODES, D), jnp.float32),
            jax.ShapeDtypeStruct((N_NODES,), jnp.float32),
            jax.ShapeDtypeStruct((2 * N_EDGES,), jnp.int32),
        ],
    )(feat, mask_row, wt, attn_row, edge_index)


def _edge_body(edge_hbm, g_hbm, en_hbm, acc_hbm, den_hbm,
               src_v, dst_v, rows_v, zero_v, en_t, den_t, acc_sh,
               sem_v, isem_v, ssem_v):
    c = lax.axis_index("c")
    s = lax.axis_index("s")
    wid = s * NC + c

    def zloop(t, _):
        i = t // (D // 16)
        k = t % (D // 16)
        zero_v[i, pl.ds(k * 16, 16)] = jnp.zeros((16,), jnp.float32)
        return 0
    lax.fori_loop(0, ZROWS * (D // 16), zloop, 0)

    def zden(t, _):
        den_t[pl.ds(t * 16, 16)] = jnp.zeros((16,), jnp.float32)
        return 0
    lax.fori_loop(0, N_NODES // 16, zden, 0)

    pltpu.sync_copy(en_hbm, en_t)

    def zcopy(j, _):
        pltpu.sync_copy(zero_v, acc_sh.at[pl.ds(s * ROWS_PER_TILE + j * ZROWS, ZROWS), :])
        return 0
    lax.fori_loop(0, ROWS_PER_TILE // ZROWS, zcopy, 0)

    plsc.subcore_barrier()

    base = wid * EPW

    def load_idx(j, q, sem):
        pltpu.async_copy(edge_hbm.at[pl.ds(base + j * CH, CH)], src_v.at[q], sem)
        pltpu.async_copy(edge_hbm.at[pl.ds(N_EDGES + base + j * CH, CH)], dst_v.at[q], sem)

    def wait_idx(j, q, sem):
        pltpu.make_async_copy(edge_hbm.at[pl.ds(base + j * CH, CH)], src_v.at[q], sem).wait()
        pltpu.make_async_copy(edge_hbm.at[pl.ds(N_EDGES + base + j * CH, CH)], dst_v.at[q], sem).wait()

    def start_gather(q, r):
        h = CH // 2
        pltpu.async_copy(g_hbm.at[src_v.at[q, pl.ds(0, h)]],
                         rows_v.at[r, pl.ds(0, h), :], sem_v.at[r])
        pltpu.async_copy(g_hbm.at[src_v.at[q, pl.ds(h, h)]],
                         rows_v.at[r, pl.ds(h, h), :], sem_v.at[r])

    def wait_gather(q, r):
        h = CH // 2
        pltpu.make_async_copy(g_hbm.at[src_v.at[q, pl.ds(0, h)]],
                              rows_v.at[r, pl.ds(0, h), :], sem_v.at[r]).wait()
        pltpu.make_async_copy(g_hbm.at[src_v.at[q, pl.ds(h, h)]],
                              rows_v.at[r, pl.ds(h, h), :], sem_v.at[r]).wait()

    def start_scatter(q, r):
        pltpu.async_copy(rows_v.at[r], acc_sh.at[dst_v.at[q]], ssem_v.at[r], add=True)

    def wait_scatter(q, r):
        pltpu.make_async_copy(rows_v.at[r], acc_sh.at[dst_v.at[q]],
                              ssem_v.at[r]).wait()

    def valu_den(q):
        sbuf, dbuf = src_v.at[q], dst_v.at[q]
        for k in range(CH // 16):
            sv = sbuf[pl.ds(k * 16, 16)]
            dv = dbuf[pl.ds(k * 16, 16)]
            ev = plsc.load_gather(en_t, [sv])
            plsc.addupdate_scatter(den_t, [dv], ev)

    pltpu.sync_copy(edge_hbm.at[pl.ds(base, CH)], src_v.at[0])
    pltpu.sync_copy(edge_hbm.at[pl.ds(N_EDGES + base, CH)], dst_v.at[0])
    load_idx(1, 1, isem_v.at[1])
    load_idx(2, 2, isem_v.at[2])
    start_gather(0, 0)
    wait_idx(1, 1, isem_v.at[1])
    start_gather(1, 1)
    load_idx(3, 3, isem_v.at[3])
    wait_gather(0, 0)
    start_scatter(0, 0)
    valu_den(0)

    def group(t, _):
        for k in range(4):
            j = 1 + 4 * t + k
            r = (1 + k) % 2
            q = (1 + k) % 4
            wait_idx(j + 1, (2 + k) % 4, isem_v.at[(2 + k) % 4])
            wait_scatter((q + 3) % 4, 1 - r)
            start_gather((2 + k) % 4, 1 - r)
            load_idx(j + 3, k, isem_v.at[k])
            wait_gather(q, r)
            start_scatter(q, r)
            valu_den(q)
        return 0
    lax.fori_loop(0, 30, group, 0)

    wait_idx(122, 2, isem_v.at[2])
    wait_scatter(0, 0)
    start_gather(2, 0)
    load_idx(124, 0, isem_v.at[0])
    wait_gather(1, 1)
    start_scatter(1, 1)
    valu_den(1)
    wait_idx(123, 3, isem_v.at[3])
    wait_scatter(1, 1)
    start_gather(3, 1)
    wait_gather(2, 0)
    start_scatter(2, 0)
    valu_den(2)
    wait_idx(124, 0, isem_v.at[0])
    wait_scatter(2, 0)
    start_gather(0, 0)
    wait_gather(3, 1)
    start_scatter(3, 1)
    valu_den(3)
    wait_scatter(3, 1)
    wait_gather(0, 0)
    start_scatter(0, 0)
    valu_den(0)
    wait_scatter(0, 0)

    pltpu.sync_copy(den_t, den_hbm.at[pl.ds(wid * DEN_STRIDE, N_NODES)])

    plsc.subcore_barrier()

    pltpu.sync_copy(acc_sh.at[pl.ds(s * ROWS_PER_TILE, ROWS_PER_TILE), :],
                    acc_hbm.at[c, pl.ds(s * ROWS_PER_TILE, ROWS_PER_TILE), :])


def _edge_phase(edge_index, g, en):
    mesh = plsc.VectorSubcoreMesh(core_axis_name="c", subcore_axis_name="s",
                                  num_cores=NC, num_subcores=NS)
    k = pl.kernel(
        _edge_body,
        out_type=[
            jax.ShapeDtypeStruct((NC, N_NODES, D), jnp.float32),
            jax.ShapeDtypeStruct((NW * DEN_STRIDE,), jnp.float32),
        ],
        mesh=mesh,
        scratch_types=[
            pltpu.VMEM((4, CH), jnp.int32),
            pltpu.VMEM((4, CH), jnp.int32),
            pltpu.VMEM((2, CH, D), jnp.float32),
            pltpu.VMEM((ZROWS, D), jnp.float32),
            pltpu.VMEM((N_NODES,), jnp.float32),
            pltpu.VMEM((N_NODES,), jnp.float32),
            pltpu.VMEM_SHARED((N_NODES, D), jnp.float32),
            pltpu.SemaphoreType.DMA((2,)),
            pltpu.SemaphoreType.DMA((4,)),
            pltpu.SemaphoreType.DMA((2,)),
        ],
        compiler_params=pltpu.CompilerParams(use_tc_tiling_on_sc=False,
                                             needs_layout_passes=False),
    )
    return k(edge_index, g, en)


def _final_body(acc_ref, den_ref, out_ref):
    num = acc_ref[0] + acc_ref[1]
    den = jnp.zeros((N_NODES,), jnp.float32)
    for w in range(NW):
        den = den + den_ref[pl.ds(w * DEN_STRIDE, N_NODES)]
    den = den[:, None]
    r = jnp.where(den > 0, num / den, 0.0)
    out_ref[...] = jnp.maximum(r, 0.0)


def _final(acc, den):
    return pl.pallas_call(
        _final_body,
        out_shape=jax.ShapeDtypeStruct((N_NODES, D), jnp.float32),
    )(acc, den)


def kernel(feat, edge_index, biclique_mask, W, attn_param):
    mask_row = biclique_mask.reshape(1, D)
    wt = W.T
    attn_row = attn_param.reshape(1, D)
    g, en, edge1d = _prep(feat, mask_row, wt, attn_row, edge_index)
    acc, den = _edge_phase(edge1d, g, en)
    return _final(acc, den)


# speedup vs baseline: 1.0454x; 1.0454x over previous
import functools

import jax
import jax.numpy as jnp
from jax import lax
from jax.experimental import pallas as pl
from jax.experimental.pallas import tpu as pltpu
from jax.experimental.pallas import tpu_sc as plsc

N_NODES = 10000
N_EDGES = 320000
D = 128

NC = 2
NS = 16
NW = NC * NS
EPW = N_EDGES // NW
CH = 80
NCHUNK = EPW // CH
ROWS_PER_TILE = N_NODES // NS
ZROWS = 25
DEN_STRIDE = 10240


def _prep_body(feat_ref, mask_ref, wt_ref, attn_ref, edge_ref,
               g_ref, en_ref, edge_out_ref):
    hm = feat_ref[...] * mask_ref[...]
    h = jnp.dot(hm, wt_ref[...], preferred_element_type=jnp.float32)
    s = jnp.sum(h * attn_ref[...], axis=1, keepdims=True)
    s = jnp.where(s > 0, s, 0.01 * s)
    en = jnp.exp(s)
    g_ref[...] = h * en
    en_ref[...] = en[:, 0]
    edge_out_ref[pl.ds(0, N_EDGES)] = edge_ref[0]
    edge_out_ref[pl.ds(N_EDGES, N_EDGES)] = edge_ref[1]


def _prep(feat, mask_row, wt, attn_row, edge_index):
    return pl.pallas_call(
        _prep_body,
        out_shape=[
            jax.ShapeDtypeStruct((N_NODES, D), jnp.float32),
            jax.ShapeDtypeStruct((N_NODES,), jnp.float32),
            jax.ShapeDtypeStruct((2 * N_EDGES,), jnp.int32),
        ],
    )(feat, mask_row, wt, attn_row, edge_index)


def _edge_body(edge_hbm, g_hbm, en_hbm, acc_hbm, den_hbm,
               src_v, dst_v, rows_v, zero_v, en_t, den_t, acc_sh,
               sem_v, isem_v, ssem_v):
    c = lax.axis_index("c")
    s = lax.axis_index("s")
    wid = s * NC + c

    base = wid * EPW

    def load_idx(j, q, sem):
        pltpu.async_copy(edge_hbm.at[pl.ds(base + j * CH, CH)], src_v.at[q], sem)
        pltpu.async_copy(edge_hbm.at[pl.ds(N_EDGES + base + j * CH, CH)], dst_v.at[q], sem)

    def wait_idx(j, q, sem):
        pltpu.make_async_copy(edge_hbm.at[pl.ds(base + j * CH, CH)], src_v.at[q], sem).wait()
        pltpu.make_async_copy(edge_hbm.at[pl.ds(N_EDGES + base + j * CH, CH)], dst_v.at[q], sem).wait()

    def start_gather(q, r):
        pltpu.async_copy(g_hbm.at[src_v.at[q]], rows_v.at[r], sem_v.at[r])

    def wait_gather(q, r):
        pltpu.make_async_copy(g_hbm.at[src_v.at[q]], rows_v.at[r], sem_v.at[r]).wait()

    def start_scatter(q, r):
        pltpu.async_copy(rows_v.at[r], acc_sh.at[dst_v.at[q]], ssem_v.at[r], add=True)

    def wait_scatter(q, r):
        pltpu.make_async_copy(rows_v.at[r], acc_sh.at[dst_v.at[q]],
                              ssem_v.at[r]).wait()

    def valu_den(q):
        sbuf, dbuf = src_v.at[q], dst_v.at[q]
        for k in range(CH // 16):
            sv = sbuf[pl.ds(k * 16, 16)]
            dv = dbuf[pl.ds(k * 16, 16)]
            ev = plsc.load_gather(en_t, [sv])
            plsc.addupdate_scatter(den_t, [dv], ev)

    def zloop(t, _):
        i = t // (D // 16)
        k = t % (D // 16)
        zero_v[i, pl.ds(k * 16, 16)] = jnp.zeros((16,), jnp.float32)
        return 0
    lax.fori_loop(0, ZROWS * (D // 16), zloop, 0)

    pltpu.async_copy(en_hbm, en_t, ssem_v.at[1])
    load_idx(0, 0, isem_v.at[0])
    load_idx(1, 1, isem_v.at[1])
    load_idx(2, 2, isem_v.at[2])

    def zcopy_start(j, _):
        pltpu.async_copy(zero_v, acc_sh.at[pl.ds(s * ROWS_PER_TILE + j * ZROWS, ZROWS), :],
                         ssem_v.at[0])
        return 0
    lax.fori_loop(0, ROWS_PER_TILE // ZROWS, zcopy_start, 0)

    def zden(t, _):
        den_t[pl.ds(t * 16, 16)] = jnp.zeros((16,), jnp.float32)
        return 0
    lax.fori_loop(0, N_NODES // 16, zden, 0)

    wait_idx(0, 0, isem_v.at[0])
    start_gather(0, 0)
    wait_idx(1, 1, isem_v.at[1])
    start_gather(1, 1)
    load_idx(3, 3, isem_v.at[3])

    def zcopy_wait(j, _):
        pltpu.make_async_copy(zero_v,
                              acc_sh.at[pl.ds(s * ROWS_PER_TILE + j * ZROWS, ZROWS), :],
                              ssem_v.at[0]).wait()
        return 0
    lax.fori_loop(0, ROWS_PER_TILE // ZROWS, zcopy_wait, 0)
    pltpu.make_async_copy(en_hbm, en_t, ssem_v.at[1]).wait()

    plsc.subcore_barrier()

    wait_gather(0, 0)
    start_scatter(0, 0)
    valu_den(0)

    def group(t, _):
        for k in range(4):
            j = 1 + 4 * t + k
            r = (1 + k) % 2
            q = (1 + k) % 4
            wait_idx(j + 1, (2 + k) % 4, isem_v.at[(2 + k) % 4])
            wait_scatter((q + 3) % 4, 1 - r)
            start_gather((2 + k) % 4, 1 - r)
            load_idx(j + 3, k, isem_v.at[k])
            wait_gather(q, r)
            start_scatter(q, r)
            valu_den(q)
        return 0
    lax.fori_loop(0, 30, group, 0)

    wait_idx(122, 2, isem_v.at[2])
    wait_scatter(0, 0)
    start_gather(2, 0)
    load_idx(124, 0, isem_v.at[0])
    wait_gather(1, 1)
    start_scatter(1, 1)
    valu_den(1)
    wait_idx(123, 3, isem_v.at[3])
    wait_scatter(1, 1)
    start_gather(3, 1)
    wait_gather(2, 0)
    start_scatter(2, 0)
    valu_den(2)
    wait_idx(124, 0, isem_v.at[0])
    wait_scatter(2, 0)
    start_gather(0, 0)
    wait_gather(3, 1)
    start_scatter(3, 1)
    valu_den(3)
    wait_scatter(3, 1)
    wait_gather(0, 0)
    start_scatter(0, 0)
    valu_den(0)
    wait_scatter(0, 0)

    pltpu.sync_copy(den_t, den_hbm.at[pl.ds(wid * DEN_STRIDE, N_NODES)])

    plsc.subcore_barrier()

    pltpu.sync_copy(acc_sh.at[pl.ds(s * ROWS_PER_TILE, ROWS_PER_TILE), :],
                    acc_hbm.at[c, pl.ds(s * ROWS_PER_TILE, ROWS_PER_TILE), :])


def _edge_phase(edge_index, g, en):
    mesh = plsc.VectorSubcoreMesh(core_axis_name="c", subcore_axis_name="s",
                                  num_cores=NC, num_subcores=NS)
    k = pl.kernel(
        _edge_body,
        out_type=[
            jax.ShapeDtypeStruct((NC, N_NODES, D), jnp.float32),
            jax.ShapeDtypeStruct((NW * DEN_STRIDE,), jnp.float32),
        ],
        mesh=mesh,
        scratch_types=[
            pltpu.VMEM((4, CH), jnp.int32),
            pltpu.VMEM((4, CH), jnp.int32),
            pltpu.VMEM((2, CH, D), jnp.float32),
            pltpu.VMEM((ZROWS, D), jnp.float32),
            pltpu.VMEM((N_NODES,), jnp.float32),
            pltpu.VMEM((N_NODES,), jnp.float32),
            pltpu.VMEM_SHARED((N_NODES, D), jnp.float32),
            pltpu.SemaphoreType.DMA((2,)),
            pltpu.SemaphoreType.DMA((4,)),
            pltpu.SemaphoreType.DMA((2,)),
        ],
        compiler_params=pltpu.CompilerParams(use_tc_tiling_on_sc=False,
                                             needs_layout_passes=False),
    )
    return k(edge_index, g, en)


def _final_body(acc_ref, den_ref, out_ref):
    num = acc_ref[0] + acc_ref[1]
    den = jnp.zeros((N_NODES,), jnp.float32)
    for w in range(NW):
        den = den + den_ref[pl.ds(w * DEN_STRIDE, N_NODES)]
    den = den[:, None]
    r = jnp.where(den > 0, num / den, 0.0)
    out_ref[...] = jnp.maximum(r, 0.0)


def _final(acc, den):
    return pl.pallas_call(
        _final_body,
        out_shape=jax.ShapeDtypeStruct((N_NODES, D), jnp.float32),
    )(acc, den)


def kernel(feat, edge_index, biclique_mask, W, attn_param):
    mask_row = biclique_mask.reshape(1, D)
    wt = W.T
    attn_row = attn_param.reshape(1, D)
    g, en, edge1d = _prep(feat, mask_row, wt, attn_row, edge_index)
    acc, den = _edge_phase(edge1d, g, en)
    return _final(acc, den)


# speedup vs baseline: 1.0489x; 1.0033x over previous
import jax
import jax.numpy as jnp
from jax import lax
from jax.experimental import pallas as pl
from jax.experimental.pallas import tpu as pltpu
from jax.experimental.pallas import tpu_sc as plsc

N_NODES = 10000
N_EDGES = 320000
D = 128

NC = 2
NS = 16
NW = NC * NS
EPW = N_EDGES // NW
CH = 80
NCHUNK = EPW // CH
ROWS_PER_TILE = N_NODES // NS
ZROWS = 25
DEN_STRIDE = 10240


def _prep_body(feat_ref, mask_ref, wt_ref, attn_ref, edge_ref,
               g_ref, en_ref, edge_out_ref):
    hm = feat_ref[...] * mask_ref[...]
    h = jnp.dot(hm, wt_ref[...], preferred_element_type=jnp.float32)
    s = jnp.sum(h * attn_ref[...], axis=1, keepdims=True)
    s = jnp.where(s > 0, s, 0.01 * s)
    en = jnp.exp(s)
    g_ref[...] = h * en
    en_ref[...] = en[:, 0]
    edge_out_ref[pl.ds(0, N_EDGES)] = edge_ref[0]
    edge_out_ref[pl.ds(N_EDGES, N_EDGES)] = edge_ref[1]


def _prep(feat, mask_row, wt, attn_row, edge_index):
    return pl.pallas_call(
        _prep_body,
        out_shape=[
            jax.ShapeDtypeStruct((N_NODES, D), jnp.float32),
            jax.ShapeDtypeStruct((N_NODES,), jnp.float32),
            jax.ShapeDtypeStruct((2 * N_EDGES,), jnp.int32),
        ],
    )(feat, mask_row, wt, attn_row, edge_index)


def _edge_body(edge_hbm, g_hbm, en_hbm, acc_hbm, den_hbm,
               src_v, dst_v, rows_v, zero_v, en_t, den_t, acc_sh,
               sem_v, isem_v, ssem_v):
    c = lax.axis_index("c")
    s = lax.axis_index("s")
    wid = s * NC + c

    base = wid * EPW

    def load_idx(j, q, sem):
        pltpu.async_copy(edge_hbm.at[pl.ds(base + j * CH, CH)], src_v.at[q], sem)
        pltpu.async_copy(edge_hbm.at[pl.ds(N_EDGES + base + j * CH, CH)], dst_v.at[q], sem)

    def wait_idx(j, q, sem):
        pltpu.make_async_copy(edge_hbm.at[pl.ds(base + j * CH, CH)], src_v.at[q], sem).wait()
        pltpu.make_async_copy(edge_hbm.at[pl.ds(N_EDGES + base + j * CH, CH)], dst_v.at[q], sem).wait()

    def start_gather(q, r):
        pltpu.async_copy(g_hbm.at[src_v.at[q]], rows_v.at[r], sem_v.at[r])

    def wait_gather(q, r):
        pltpu.make_async_copy(g_hbm.at[src_v.at[q]], rows_v.at[r], sem_v.at[r]).wait()

    def start_scatter(q, r):
        pltpu.async_copy(rows_v.at[r], acc_sh.at[dst_v.at[q]], ssem_v.at[r], add=True)

    def wait_scatter(q, r):
        pltpu.make_async_copy(rows_v.at[r], acc_sh.at[dst_v.at[q]],
                              ssem_v.at[r]).wait()

    def valu_den(q):
        sbuf, dbuf = src_v.at[q], dst_v.at[q]
        for k in range(CH // 16):
            sv = sbuf[pl.ds(k * 16, 16)]
            dv = dbuf[pl.ds(k * 16, 16)]
            ev = plsc.load_gather(en_t, [sv])
            plsc.addupdate_scatter(den_t, [dv], ev)

    def zloop(t, _):
        i = t // (D // 16)
        k = t % (D // 16)
        zero_v[i, pl.ds(k * 16, 16)] = jnp.zeros((16,), jnp.float32)
        return 0
    lax.fori_loop(0, ZROWS * (D // 16), zloop, 0)

    pltpu.async_copy(en_hbm, en_t, ssem_v.at[1])
    load_idx(0, 0, isem_v.at[0])
    load_idx(1, 1, isem_v.at[1])
    load_idx(2, 2, isem_v.at[2])

    def zcopy_start(j, _):
        pltpu.async_copy(zero_v, acc_sh.at[pl.ds(s * ROWS_PER_TILE + j * ZROWS, ZROWS), :],
                         ssem_v.at[0])
        return 0
    lax.fori_loop(0, ROWS_PER_TILE // ZROWS, zcopy_start, 0)

    def zden(t, _):
        den_t[pl.ds(t * 16, 16)] = jnp.zeros((16,), jnp.float32)
        return 0
    lax.fori_loop(0, N_NODES // 16, zden, 0)

    wait_idx(0, 0, isem_v.at[0])
    start_gather(0, 0)
    wait_idx(1, 1, isem_v.at[1])
    start_gather(1, 1)
    load_idx(3, 3, isem_v.at[3])

    def zcopy_wait(j, _):
        pltpu.make_async_copy(zero_v,
                              acc_sh.at[pl.ds(s * ROWS_PER_TILE + j * ZROWS, ZROWS), :],
                              ssem_v.at[0]).wait()
        return 0
    lax.fori_loop(0, ROWS_PER_TILE // ZROWS, zcopy_wait, 0)
    pltpu.make_async_copy(en_hbm, en_t, ssem_v.at[1]).wait()

    plsc.subcore_barrier()

    wait_gather(0, 0)
    start_scatter(0, 0)
    valu_den(0)

    def group(t, _):
        for k in range(4):
            j = 1 + 4 * t + k
            r = (1 + k) % 2
            q = (1 + k) % 4
            wait_idx(j + 1, (2 + k) % 4, isem_v.at[(2 + k) % 4])
            wait_scatter((q + 3) % 4, 1 - r)
            start_gather((2 + k) % 4, 1 - r)
            load_idx(j + 3, k, isem_v.at[k])
            wait_gather(q, r)
            start_scatter(q, r)
            valu_den(q)
        return 0
    lax.fori_loop(0, 30, group, 0)

    wait_idx(122, 2, isem_v.at[2])
    wait_scatter(0, 0)
    start_gather(2, 0)
    load_idx(124, 0, isem_v.at[0])
    wait_gather(1, 1)
    start_scatter(1, 1)
    valu_den(1)
    wait_idx(123, 3, isem_v.at[3])
    wait_scatter(1, 1)
    start_gather(3, 1)
    wait_gather(2, 0)
    start_scatter(2, 0)
    valu_den(2)
    wait_idx(124, 0, isem_v.at[0])
    wait_scatter(2, 0)
    start_gather(0, 0)
    wait_gather(3, 1)
    start_scatter(3, 1)
    valu_den(3)
    wait_scatter(3, 1)
    wait_gather(0, 0)
    start_scatter(0, 0)
    valu_den(0)
    wait_scatter(0, 0)

    pltpu.sync_copy(den_t, den_hbm.at[pl.ds(wid * DEN_STRIDE, N_NODES)])

    plsc.subcore_barrier()

    pltpu.sync_copy(acc_sh.at[pl.ds(s * ROWS_PER_TILE, ROWS_PER_TILE), :],
                    acc_hbm.at[c, pl.ds(s * ROWS_PER_TILE, ROWS_PER_TILE), :])


def _edge_phase(edge_index, g, en):
    mesh = plsc.VectorSubcoreMesh(core_axis_name="c", subcore_axis_name="s",
                                  num_cores=NC, num_subcores=NS)
    k = pl.kernel(
        _edge_body,
        out_type=[
            jax.ShapeDtypeStruct((NC, N_NODES, D), jnp.float32),
            jax.ShapeDtypeStruct((NW * DEN_STRIDE,), jnp.float32),
        ],
        mesh=mesh,
        scratch_types=[
            pltpu.VMEM((4, CH), jnp.int32),
            pltpu.VMEM((4, CH), jnp.int32),
            pltpu.VMEM((2, CH, D), jnp.float32),
            pltpu.VMEM((ZROWS, D), jnp.float32),
            pltpu.VMEM((N_NODES,), jnp.float32),
            pltpu.VMEM((N_NODES,), jnp.float32),
            pltpu.VMEM_SHARED((N_NODES, D), jnp.float32),
            pltpu.SemaphoreType.DMA((2,)),
            pltpu.SemaphoreType.DMA((4,)),
            pltpu.SemaphoreType.DMA((2,)),
        ],
        compiler_params=pltpu.CompilerParams(use_tc_tiling_on_sc=False,
                                             needs_layout_passes=False),
    )
    return k(edge_index, g, en)


def _final_body(acc_ref, den_ref, out_ref):
    num = acc_ref[0] + acc_ref[1]
    den = jnp.zeros((N_NODES,), jnp.float32)
    for w in range(NW):
        den = den + den_ref[pl.ds(w * DEN_STRIDE, N_NODES)]
    den = den[:, None]
    r = jnp.where(den > 0, num / den, 0.0)
    out_ref[...] = jnp.maximum(r, 0.0)


def _final(acc, den):
    return pl.pallas_call(
        _final_body,
        out_shape=jax.ShapeDtypeStruct((N_NODES, D), jnp.float32),
    )(acc, den)


def kernel(feat, edge_index, biclique_mask, W, attn_param):
    mask_row = biclique_mask.reshape(1, D)
    wt = W.T
    attn_row = attn_param.reshape(1, D)
    g, en, edge1d = _prep(feat, mask_row, wt, attn_row, edge_index)
    acc, den = _edge_phase(edge1d, g, en)
    return _final(acc, den)
